# baseline (speedup 1.0000x reference)
import functools

import jax
import jax.numpy as jnp
from jax.experimental import pallas as pl
from jax.experimental.pallas import tpu as pltpu

D = 4096
N = 64
BT = 512


def _body(x_ref, p_ref, g_ref, o_ref, *, scale):
    acc = jax.lax.dot_general(
        x_ref[...], p_ref[...],
        dimension_numbers=(((1,), (1,)), ((), ())),
        preferred_element_type=jnp.float32,
    )
    o_ref[...] = jnp.maximum(acc * scale - g_ref[...], 0.0)


def kernel(x, proto_k, gate):
    t, d = x.shape
    n = proto_k.shape[0]
    scale = 1.0 / (d ** 0.5)
    gate2d = gate.reshape(1, n)
    grid = (t // BT,)
    return pl.pallas_call(
        functools.partial(_body, scale=scale),
        grid=grid,
        in_specs=[
            pl.BlockSpec((BT, d), lambda i: (i, 0)),
            pl.BlockSpec((n, d), lambda i: (0, 0)),
            pl.BlockSpec((1, n), lambda i: (0, 0)),
        ],
        out_specs=pl.BlockSpec((BT, n), lambda i: (i, 0)),
        out_shape=jax.ShapeDtypeStruct((t, n), jnp.float32),
        compiler_params=pltpu.CompilerParams(
            dimension_semantics=("parallel",),
        ),
    )(x, proto_k, gate2d)
